# Initial kernel scaffold; baseline (speedup 1.0000x reference)
import jax
import jax.numpy as jnp
from jax.experimental import pallas as pl


def kernel(x, W1_self, W1_neigh, b1, W2_self, W2_neigh, b2, Wp, bp, block1_edge_index, block2_edge_index, pos_edge_index, neg_edge_index):
    raise NotImplementedError("write your pallas kernel here")


# speedup vs baseline: 8.0766x; 8.0766x over previous
import functools

import jax
import jax.numpy as jnp
from jax import lax
from jax.experimental import pallas as pl
from jax.experimental.pallas import tpu as pltpu
from jax.experimental.pallas import tpu_sc as plsc

N = 10000
D = 128
E = 320000

NC = 2
NS = 16
NW = NC * NS
EPW = E // NW
CHUNK = 80
NCHUNK = EPW // CHUNK
NPAD = 10240
RPT = NPAD // NS

_SC_MESH = plsc.VectorSubcoreMesh(
    core_axis_name="c", subcore_axis_name="s", num_cores=NC, num_subcores=NS)


@functools.partial(
    pl.kernel,
    out_type=[jax.ShapeDtypeStruct((NC * NPAD, D), jnp.float32)],
    mesh=_SC_MESH,
    scratch_types=[
        pltpu.VMEM((CHUNK,), jnp.int32),
        pltpu.VMEM((CHUNK,), jnp.int32),
        pltpu.VMEM((CHUNK, D), jnp.float32),
        pltpu.VMEM_SHARED((NPAD, D), jnp.float32),
        pltpu.SemaphoreType.DMA,
    ],
)
def _sc_rows(table_hbm, src_hbm, dst_hbm, out_hbm,
             src_v, dst_v, rows_v, acc_sh, sem):
    cid = lax.axis_index("c")
    sid = lax.axis_index("s")
    wid = sid * NC + cid

    zf = jnp.zeros((16,), jnp.float32)

    def _fill_zero(i, carry):
        for j in range(D // 16):
            rows_v[i, pl.ds(j * 16, 16)] = zf
        return carry

    lax.fori_loop(0, CHUNK, _fill_zero, 0)
    lo = sid * RPT
    for r in range(RPT // CHUNK):
        pltpu.sync_copy(rows_v, acc_sh.at[pl.ds(lo + r * CHUNK, CHUNK)])
    plsc.subcore_barrier()

    base = wid * EPW

    def _edge_chunk(j, carry):
        off = base + j * CHUNK
        pltpu.sync_copy(src_hbm.at[pl.ds(off, CHUNK)], src_v)
        pltpu.sync_copy(dst_hbm.at[pl.ds(off, CHUNK)], dst_v)
        pltpu.async_copy(table_hbm.at[src_v], rows_v, sem).wait()
        pltpu.sync_copy(rows_v, acc_sh.at[dst_v], add=True)
        return carry

    lax.fori_loop(0, NCHUNK, _edge_chunk, 0)
    plsc.subcore_barrier()

    def _wb(r, carry):
        ro = lo + r * CHUNK
        pltpu.sync_copy(acc_sh.at[pl.ds(ro, CHUNK)], rows_v)
        pltpu.sync_copy(rows_v, out_hbm.at[pl.ds(cid * NPAD + ro, CHUNK)])
        return carry

    lax.fori_loop(0, RPT // CHUNK, _wb, 0)


@functools.partial(
    pl.kernel,
    out_type=[
        jax.ShapeDtypeStruct((NC * NPAD,), jnp.float32),
        jax.ShapeDtypeStruct((NC * NPAD,), jnp.float32),
    ],
    mesh=_SC_MESH,
    scratch_types=[
        pltpu.VMEM((CHUNK,), jnp.int32),
        pltpu.VMEM((CHUNK,), jnp.float32),
        pltpu.VMEM((RPT,), jnp.float32),
        pltpu.VMEM_SHARED((NPAD,), jnp.float32),
        pltpu.VMEM_SHARED((NPAD,), jnp.float32),
    ],
)
def _sc_degrees(dst1_hbm, dst2_hbm, deg1_out, deg2_out,
                dst_v, ones_v, buf_v, deg1_sh, deg2_sh):
    cid = lax.axis_index("c")
    sid = lax.axis_index("s")
    wid = sid * NC + cid

    zf = jnp.zeros((16,), jnp.float32)
    onef = jnp.full((16,), 1.0, jnp.float32)

    def _fill(i, carry):
        buf_v[pl.ds(i * 16, 16)] = zf
        return carry

    lax.fori_loop(0, RPT // 16, _fill, 0)
    for k in range(CHUNK // 16):
        ones_v[pl.ds(k * 16, 16)] = onef

    lo = sid * RPT
    pltpu.sync_copy(buf_v, deg1_sh.at[pl.ds(lo, RPT)])
    pltpu.sync_copy(buf_v, deg2_sh.at[pl.ds(lo, RPT)])
    plsc.subcore_barrier()

    base = wid * EPW

    def _edge_chunk(j, carry):
        off = base + j * CHUNK
        pltpu.sync_copy(dst1_hbm.at[pl.ds(off, CHUNK)], dst_v)
        pltpu.sync_copy(ones_v, deg1_sh.at[dst_v], add=True)
        pltpu.sync_copy(dst2_hbm.at[pl.ds(off, CHUNK)], dst_v)
        pltpu.sync_copy(ones_v, deg2_sh.at[dst_v], add=True)
        return carry

    lax.fori_loop(0, NCHUNK, _edge_chunk, 0)
    plsc.subcore_barrier()

    pltpu.sync_copy(deg1_sh.at[pl.ds(lo, RPT)], buf_v)
    pltpu.sync_copy(buf_v, deg1_out.at[pl.ds(cid * NPAD + lo, RPT)])
    pltpu.sync_copy(deg2_sh.at[pl.ds(lo, RPT)], buf_v)
    pltpu.sync_copy(buf_v, deg2_out.at[pl.ds(cid * NPAD + lo, RPT)])


SPW = 2000
NSCHUNK = EPW // SPW


@functools.partial(
    pl.kernel,
    out_type=[
        jax.ShapeDtypeStruct((E,), jnp.float32),
        jax.ShapeDtypeStruct((E,), jnp.float32),
    ],
    mesh=_SC_MESH,
    scratch_types=[
        pltpu.VMEM((N,), jnp.float32),
        pltpu.VMEM((N,), jnp.float32),
        pltpu.VMEM((SPW,), jnp.int32),
        pltpu.VMEM((SPW,), jnp.int32),
        pltpu.VMEM((SPW,), jnp.float32),
    ],
    compiler_params=pltpu.CompilerParams(needs_layout_passes=False),
)
def _sc_predict(su_hbm, sv_hbm, pu_hbm, pv_hbm, nu_hbm, nv_hbm,
                pos_out, neg_out, su_v, sv_v, iu_v, iv_v, ob_v):
    cid = lax.axis_index("c")
    sid = lax.axis_index("s")
    wid = sid * NC + cid
    base = wid * EPW

    pltpu.sync_copy(su_hbm, su_v)
    pltpu.sync_copy(sv_hbm, sv_v)

    for u_hbm, v_hbm, out_hbm in ((pu_hbm, pv_hbm, pos_out),
                                  (nu_hbm, nv_hbm, neg_out)):
        def _chunk(j, carry, u_hbm=u_hbm, v_hbm=v_hbm, out_hbm=out_hbm):
            off = base + j * SPW
            pltpu.sync_copy(u_hbm.at[pl.ds(off, SPW)], iu_v)
            pltpu.sync_copy(v_hbm.at[pl.ds(off, SPW)], iv_v)

            def _vec(k, c2):
                iu = iu_v[pl.ds(k * 16, 16)]
                iv = iv_v[pl.ds(k * 16, 16)]
                a = plsc.load_gather(su_v, [iu])
                b = plsc.load_gather(sv_v, [iv])
                ob_v[pl.ds(k * 16, 16)] = a + b
                return c2

            lax.fori_loop(0, SPW // 16, _vec, 0)
            pltpu.sync_copy(ob_v, out_hbm.at[pl.ds(off, SPW)])
            return carry

        lax.fori_loop(0, NSCHUNK, _chunk, 0)


RB = 1000


def _tc_layer1(x_b, agg_b, deg_b, ws_b, wn_b, b_b, o_b):
    a = agg_b[0] + agg_b[1]
    dg = deg_b[0] + deg_b[1]
    hn = a * (1.0 / jnp.maximum(dg, 1.0))
    h = (jnp.dot(x_b[...], ws_b[...], preferred_element_type=jnp.float32)
         + jnp.dot(hn, wn_b[...], preferred_element_type=jnp.float32)
         + b_b[...])
    o_b[...] = jnp.maximum(h, 0.0)


def _tc_layer2(x_b, agg_b, deg_b, ws_b, wn_b, b_b, wp_b, bp_b, s_b):
    a = agg_b[0] + agg_b[1]
    dg = deg_b[0] + deg_b[1]
    hn = a * (1.0 / jnp.maximum(dg, 1.0))
    h = (jnp.dot(x_b[...], ws_b[...], preferred_element_type=jnp.float32)
         + jnp.dot(hn, wn_b[...], preferred_element_type=jnp.float32)
         + b_b[...])
    s_b[...] = jnp.dot(h, wp_b[...], preferred_element_type=jnp.float32) + bp_b[...]


def _row_spec():
    return pl.BlockSpec((RB, D), lambda i: (i, 0))


def _shared_specs():
    return [
        pl.BlockSpec((NC, RB, D), lambda i: (0, i, 0)),
        pl.BlockSpec((NC, RB, 1), lambda i: (0, i, 0)),
        pl.BlockSpec((D, D), lambda i: (0, 0)),
        pl.BlockSpec((D, D), lambda i: (0, 0)),
        pl.BlockSpec((1, D), lambda i: (0, 0)),
    ]


def kernel(x, W1_self, W1_neigh, b1, W2_self, W2_neigh, b2, Wp, bp,
           block1_edge_index, block2_edge_index, pos_edge_index,
           neg_edge_index):
    f32 = jnp.float32
    src1, dst1 = block1_edge_index[0], block1_edge_index[1]
    src2, dst2 = block2_edge_index[0], block2_edge_index[1]
    pu, pv = pos_edge_index[0], pos_edge_index[1]
    nu, nv = neg_edge_index[0], neg_edge_index[1]

    deg1, deg2 = _sc_degrees(dst1, dst2)
    deg1 = deg1.reshape(NC, NPAD, 1)
    deg2 = deg2.reshape(NC, NPAD, 1)

    (agg1,) = _sc_rows(x, src1, dst1)
    agg1 = agg1.reshape(NC, NPAD, D)

    h1 = pl.pallas_call(
        _tc_layer1,
        grid=(N // RB,),
        in_specs=[_row_spec()] + _shared_specs(),
        out_specs=_row_spec(),
        out_shape=jax.ShapeDtypeStruct((N, D), f32),
    )(x, agg1, deg1, W1_self, W1_neigh, b1.reshape(1, D))

    (agg2,) = _sc_rows(h1, src2, dst2)
    agg2 = agg2.reshape(NC, NPAD, D)

    wp2 = jnp.concatenate([Wp[:D], Wp[D:]], axis=1)
    bp2 = jnp.concatenate([bp, jnp.zeros((1,), f32)]).reshape(1, 2)

    s = pl.pallas_call(
        _tc_layer2,
        grid=(N // RB,),
        in_specs=[_row_spec()] + _shared_specs() + [
            pl.BlockSpec((D, 2), lambda i: (0, 0)),
            pl.BlockSpec((1, 2), lambda i: (0, 0)),
        ],
        out_specs=pl.BlockSpec((RB, 2), lambda i: (i, 0)),
        out_shape=jax.ShapeDtypeStruct((N, 2), f32),
    )(h1, agg2, deg2, W2_self, W2_neigh, b2.reshape(1, D), wp2, bp2)

    st = s.T
    su = st[0]
    sv = st[1]

    pos_score, neg_score = _sc_predict(su, sv, pu, pv, nu, nv)
    return pos_score.reshape(E, 1), neg_score.reshape(E, 1)
